# Initial kernel scaffold; baseline (speedup 1.0000x reference)
import jax
import jax.numpy as jnp
from jax.experimental import pallas as pl


def kernel(text, emb_table, w_ih, w_hh, b_ih, b_hh, fc_w, fc_b):
    raise NotImplementedError("write your pallas kernel here")


# speedup vs baseline: 2.9837x; 2.9837x over previous
import functools

import jax
import jax.numpy as jnp
from jax.experimental import pallas as pl
from jax.experimental.pallas import tpu as pltpu
from jax.experimental.pallas import tpu_sc as plsc

VOCAB = 1000
EMB = 64
HID = 32
G4 = 4 * HID
NCLS = 10
B = 4096
L = 200

GATHER_WINDOW = 128


def _project_kernel(emb_ref, wih_ref, bias_ref, p_ref):
    p_ref[...] = (
        jnp.dot(emb_ref[...], wih_ref[...], preferred_element_type=jnp.float32)
        + bias_ref[...]
    )


def _project_table(emb_table, wih_t, bias):
    return pl.pallas_call(
        _project_kernel,
        out_shape=jax.ShapeDtypeStruct((VOCAB, G4), jnp.float32),
    )(emb_table, wih_t, bias)


def _sc_gather(table, idx_flat):
    n_idx = idx_flat.shape[0]
    d = table.shape[1]
    mesh = plsc.VectorSubcoreMesh(core_axis_name="c", subcore_axis_name="s")
    n_workers = 32
    per_w = n_idx // n_workers
    n_chunks = per_w // GATHER_WINDOW
    w = GATHER_WINDOW

    @functools.partial(
        pl.kernel,
        out_type=jax.ShapeDtypeStruct((n_idx, d), table.dtype),
        mesh=mesh,
        scratch_types=[
            pltpu.VMEM((per_w,), jnp.int32),
            pltpu.VMEM((w, d), table.dtype),
            pltpu.VMEM((w, d), table.dtype),
            pltpu.SemaphoreType.DMA,
            pltpu.SemaphoreType.DMA,
        ],
    )
    def gather_kernel(table_hbm, idx_hbm, out_hbm, idx_v, rows0, rows1,
                      gsem0, gsem1):
        wid = jax.lax.axis_index("s") * 2 + jax.lax.axis_index("c")
        base = wid * per_w
        pltpu.sync_copy(idx_hbm.at[pl.ds(base, per_w)], idx_v)

        def start_gather(chunk, rows, sem):
            pltpu.async_copy(
                table_hbm.at[idx_v.at[pl.ds(chunk * w, w)]], rows, sem)

        def wait_gather(rows, sem):
            pltpu.make_async_copy(
                table_hbm.at[idx_v.at[pl.ds(0, w)]], rows, sem).wait()

        start_gather(0, rows0, gsem0)

        @pl.loop(0, n_chunks, step=2)
        def _(ck):
            start_gather(ck + 1, rows1, gsem1)
            wait_gather(rows0, gsem0)
            pltpu.sync_copy(rows0, out_hbm.at[pl.ds(base + ck * w, w)])

            @pl.when(ck + 2 < n_chunks)
            def _():
                start_gather(ck + 2, rows0, gsem0)

            wait_gather(rows1, gsem1)
            pltpu.sync_copy(rows1, out_hbm.at[pl.ds(base + (ck + 1) * w, w)])

    return gather_kernel(table, idx_flat)


def _lstm_step_kernel(gx_ref, whh_ref, fcw_ref, fcb_ref,
                      out_ref, h_ref, c_ref):
    t = pl.program_id(0)

    @pl.when(t == 0)
    def _():
        h_ref[...] = jnp.zeros_like(h_ref)
        c_ref[...] = jnp.zeros_like(c_ref)

    h = h_ref[...]
    gates = gx_ref[0] + jnp.dot(
        h, whh_ref[...], preferred_element_type=jnp.float32)
    i = jax.nn.sigmoid(gates[:, 0 * HID:1 * HID])
    f = jax.nn.sigmoid(gates[:, 1 * HID:2 * HID])
    g = jnp.tanh(gates[:, 2 * HID:3 * HID])
    o = jax.nn.sigmoid(gates[:, 3 * HID:4 * HID])
    c = f * c_ref[...] + i * g
    h = o * jnp.tanh(c)
    c_ref[...] = c
    h_ref[...] = h

    @pl.when(t == L - 1)
    def _():
        out_ref[...] = (
            jnp.dot(h, fcw_ref[...], preferred_element_type=jnp.float32)
            + fcb_ref[...]
        )


def _tc_lstm(gx, whh_t, fcw_t, fcb):
    return pl.pallas_call(
        _lstm_step_kernel,
        grid=(L,),
        in_specs=[
            pl.BlockSpec((1, B, G4), lambda t: (t, 0, 0)),
            pl.BlockSpec((HID, G4), lambda t: (0, 0)),
            pl.BlockSpec((HID, NCLS), lambda t: (0, 0)),
            pl.BlockSpec((1, NCLS), lambda t: (0, 0)),
        ],
        out_specs=pl.BlockSpec((B, NCLS), lambda t: (0, 0)),
        out_shape=jax.ShapeDtypeStruct((B, NCLS), jnp.float32),
        scratch_shapes=[
            pltpu.VMEM((B, HID), jnp.float32),
            pltpu.VMEM((B, HID), jnp.float32),
        ],
    )(gx, whh_t, fcw_t, fcb)


@jax.jit
def kernel(text, emb_table, w_ih, w_hh, b_ih, b_hh, fc_w, fc_b):
    idx_flat = text.T.astype(jnp.int32).reshape(L * B)

    wih_t = w_ih.T
    bias = (b_ih + b_hh).reshape(1, G4)
    proj = _project_table(emb_table, wih_t, bias)

    gx = _sc_gather(proj, idx_flat).reshape(L, B, G4)

    whh_t = w_hh.T
    fcw_t = fc_w.T
    fcb = fc_b.reshape(1, NCLS)
    return _tc_lstm(gx, whh_t, fcw_t, fcb)
